# baseline (speedup 1.0000x reference)
import functools

import jax
import jax.numpy as jnp
import numpy as np
from jax import lax
from jax.experimental import pallas as pl
from jax.experimental.pallas import tpu as pltpu
from jax.experimental.pallas import tpu_sc as plsc

EPS = 0.05
N_TPL = 8
TPL_N = 16
TPL_W = N_TPL * TPL_N

_NC = 2
_NS = 16
_NW = _NC * _NS

_CH = 125
_IB = 16


def _padded_n(N):
    q = _NS * 8
    return ((N + q - 1) // q) * q


def _make_hist(E, N):
    rows = E // _CH
    rpw = rows // _NW
    NP = _padded_n(N)
    npw = NP // _NS
    mesh = plsc.VectorSubcoreMesh(core_axis_name="c", subcore_axis_name="s")

    @functools.partial(
        pl.kernel, mesh=mesh,
        out_type=jax.ShapeDtypeStruct((2 * NP, TPL_W), jnp.float32),
        scratch_types=[
            pltpu.VMEM((rpw, _CH), jnp.int32),
            pltpu.VMEM((_CH, TPL_W), jnp.float32),
            pltpu.VMEM_SHARED((NP, TPL_W), jnp.float32),
            pltpu.SemaphoreType.DMA,
        ],
    )
    def hist(dst_hbm, ones_hbm, zeros_hbm, out_hbm, dst_v, ones_v, acc_sh,
             ssem):
        cid = lax.axis_index("c")
        sid = lax.axis_index("s")
        wid = cid * _NS + sid
        pltpu.sync_copy(zeros_hbm.at[pl.ds(0, npw)],
                        acc_sh.at[pl.ds(sid * npw, npw)])
        pltpu.sync_copy(ones_hbm, ones_v)
        pltpu.sync_copy(dst_hbm.at[pl.ds(wid * rpw, rpw)], dst_v)
        plsc.subcore_barrier()

        def body(c, carry):
            @pl.when(c < rpw)
            def _():
                pltpu.async_copy(ones_v, acc_sh.at[dst_v.at[c]], ssem,
                                 add=True)

            @pl.when(c >= 8)
            def _():
                pltpu.make_async_copy(ones_v, acc_sh.at[dst_v.at[c - 8]],
                                      ssem).wait()

            return carry

        lax.fori_loop(0, rpw + 8, body, 0)
        plsc.subcore_barrier()
        pltpu.sync_copy(acc_sh.at[pl.ds(sid * npw, npw)],
                        out_hbm.at[pl.ds(cid * NP + sid * npw, npw)])

    return hist


def _make_seg(E, N):
    W = TPL_W
    rows = E // _CH
    rpw = rows // _NW
    NP = _padded_n(N)
    npw = NP // _NS
    mesh = plsc.VectorSubcoreMesh(core_axis_name="c", subcore_axis_name="s")

    assert rpw % _IB == 0

    @functools.partial(
        pl.kernel, mesh=mesh,
        out_type=jax.ShapeDtypeStruct((2 * NP, W), jnp.float32),
        scratch_types=[
            pltpu.VMEM((2, _IB, _CH), jnp.int32),
            pltpu.VMEM((2, _IB, _CH), jnp.int32),
            pltpu.VMEM((2, _CH, W), jnp.float32),
            pltpu.VMEM_SHARED((NP, W), jnp.float32),
            pltpu.SemaphoreType.DMA,
            pltpu.SemaphoreType.DMA,
            pltpu.SemaphoreType.DMA,
        ],
    )
    def seg(vals_hbm, src_hbm, dst_hbm, zeros_hbm, out_hbm,
            src_v, dst_v, rows_v, acc_sh, gsem, isem, ssem):
        cid = lax.axis_index("c")
        sid = lax.axis_index("s")
        wid = cid * _NS + sid
        base = wid * rpw
        pltpu.sync_copy(src_hbm.at[pl.ds(base, _IB)], src_v.at[0])
        pltpu.sync_copy(dst_hbm.at[pl.ds(base, _IB)], dst_v.at[0])
        pltpu.sync_copy(zeros_hbm.at[pl.ds(0, npw)],
                        acc_sh.at[pl.ds(sid * npw, npw)])
        plsc.subcore_barrier()

        def body(c, carry):
            p = c % 2
            j = c // _IB
            jp = j % 2
            cm = c - 1
            pm = 1 - p
            jm = cm // _IB
            c2 = c - 2
            j2 = c2 // _IB

            @pl.when((c >= 1) & (c <= rpw))
            def _():
                pltpu.make_async_copy(
                    vals_hbm.at[src_v.at[jm % 2].at[cm % _IB]],
                    rows_v.at[pm], gsem).wait()

            @pl.when((c >= 2) & (c2 < rpw))
            def _():
                pltpu.make_async_copy(
                    rows_v.at[p],
                    acc_sh.at[dst_v.at[j2 % 2].at[c2 % _IB]], ssem).wait()

            @pl.when((c % _IB == 0) & (c > 0) & (c < rpw))
            def _():
                pltpu.make_async_copy(
                    src_hbm.at[pl.ds(base + j * _IB, _IB)],
                    src_v.at[jp], isem).wait()
                pltpu.make_async_copy(
                    dst_hbm.at[pl.ds(base + j * _IB, _IB)],
                    dst_v.at[jp], isem).wait()

            @pl.when(c < rpw)
            def _():
                pltpu.async_copy(
                    vals_hbm.at[src_v.at[jp].at[c % _IB]],
                    rows_v.at[p], gsem)

            @pl.when((c >= 1) & (c <= rpw))
            def _():
                pltpu.async_copy(rows_v.at[pm],
                                 acc_sh.at[dst_v.at[jm % 2].at[cm % _IB]],
                                 ssem, add=True)

            @pl.when((c % _IB == 1) & (c - 1 + _IB < rpw))
            def _():
                off = base + (j + 1) * _IB
                pltpu.async_copy(src_hbm.at[pl.ds(off, _IB)],
                                 src_v.at[1 - jp], isem)
                pltpu.async_copy(dst_hbm.at[pl.ds(off, _IB)],
                                 dst_v.at[1 - jp], isem)

            return carry

        lax.fori_loop(0, rpw + 2, body, 0)
        plsc.subcore_barrier()
        pltpu.sync_copy(acc_sh.at[pl.ds(sid * npw, npw)],
                        out_hbm.at[pl.ds(cid * NP + sid * npw, npw)])

    return seg


def _blockmax(row):
    parts = []
    for k in range(N_TPL):
        m = jnp.max(row[:, k * TPL_N:(k + 1) * TPL_N], axis=1, keepdims=True)
        parts.append(jnp.broadcast_to(m, (1, TPL_N)))
    return jnp.concatenate(parts, axis=1)


def _bblock():
    r = lax.broadcasted_iota(jnp.int32, (TPL_W, TPL_W), 0) // TPL_N
    c = lax.broadcasted_iota(jnp.int32, (TPL_W, TPL_W), 1) // TPL_N
    return (r == c).astype(jnp.float32)


def _round_math(h, deg_raw, A, alpha, consts, loga, invN, want_T=True):
    FT, Fsq, colconst, C2bd, logb = consts
    hsq = jnp.sum(h * h, axis=1, keepdims=True)
    M = hsq + Fsq - 2.0 * jnp.dot(h, FT, preferred_element_type=jnp.float32)
    cC = deg_raw * invN + colconst
    tens = cC - 2.0 * jnp.dot(A, C2bd, preferred_element_type=jnp.float32)
    G = (1.0 - alpha) * M + alpha * tens
    if not want_T:
        return None, G
    Bb = _bblock()
    mb = _blockmax(jnp.max(jnp.abs(G), axis=0, keepdims=True))
    negGn_eps = G * (-1.0 / (EPS * (mb + 1e-8)))
    gdiv = jnp.zeros((1, TPL_W), jnp.float32)
    fdiv = None
    for _ in range(5):
        X = gdiv + negGn_eps
        xm_b = _blockmax(jnp.max(X, axis=0, keepdims=True))
        Ex = jnp.exp(X - xm_b)
        Sb = jnp.dot(Ex, Bb, preferred_element_type=jnp.float32)
        fdiv = loga - xm_b - jnp.log(Sb)
        Y = fdiv + negGn_eps
        ym = jnp.max(Y, axis=0, keepdims=True)
        Sg = jnp.sum(jnp.exp(Y - ym), axis=0, keepdims=True)
        gdiv = logb - ym - jnp.log(Sg)
    T = jnp.exp(fdiv + gdiv + negGn_eps)
    return T, G


def _k1_body(x_ref, w_ref, cnt_ref, h0_ref, hs_ref, dr_ref, dinv_ref, idg_ref):
    N = x_ref.shape[0]
    H = w_ref.shape[1]
    NP = cnt_ref.shape[0] // 2
    h0 = jnp.dot(x_ref[...], w_ref[...], preferred_element_type=jnp.float32)
    cnt = cnt_ref[...]
    deg_raw = cnt[:N, 0:1] + cnt[NP:NP + N, 0:1]
    deg = deg_raw + 1.0
    dinv = lax.rsqrt(deg)
    h0_ref[...] = h0
    hs = h0 * dinv
    hs_ref[...] = jnp.concatenate(
        [hs, jnp.zeros((N, TPL_W - H), jnp.float32)], axis=1)
    dr_ref[...] = deg_raw
    dinv_ref[...] = dinv
    idg_ref[...] = 1.0 / deg


def _k2_body(loga, invN, h0_ref, S_ref, dr_ref, dinv_ref, idg_ref, bg_ref,
             bflat_ref, alpha_ref, FT_ref, Fsq_ref, cc_ref, C2bd_ref,
             logb_ref, h_ref, T_ref):
    N = h0_ref.shape[0]
    H = h0_ref.shape[1]
    NP = S_ref.shape[0] // 2
    S = S_ref[:N, :H] + S_ref[NP:NP + N, :H]
    h = dinv_ref[...] * S + h0_ref[...] * idg_ref[...] + bg_ref[...]
    h_ref[...] = h
    deg_raw = dr_ref[...]
    alpha = alpha_ref[0, 0]
    consts = (FT_ref[...], Fsq_ref[...], cc_ref[...], C2bd_ref[...],
              logb_ref[...])
    A0 = deg_raw * invN * bflat_ref[...]
    T, _ = _round_math(h, deg_raw, A0, alpha, consts, loga, invN, want_T=True)
    T_ref[...] = T


def _kround_body(loga, invN, h_ref, A_ref, dr_ref, alpha_ref, FT_ref,
                 Fsq_ref, cc_ref, C2bd_ref, logb_ref, T_ref):
    N = h_ref.shape[0]
    NP = A_ref.shape[0] // 2
    A = A_ref[:N, :] + A_ref[NP:NP + N, :]
    alpha = alpha_ref[0, 0]
    consts = (FT_ref[...], Fsq_ref[...], cc_ref[...], C2bd_ref[...],
              logb_ref[...])
    T, _ = _round_math(h_ref[...], dr_ref[...], A, alpha, consts, loga, invN,
                       want_T=True)
    T_ref[...] = T


def _kfinal_body(loga, invN, h_ref, A_ref, Tin_ref, dr_ref, alpha_ref,
                 FT_ref, Fsq_ref, cc_ref, C2bd_ref, logb_ref,
                 wlin_ref, blin_ref, out_ref):
    N = h_ref.shape[0]
    NP = A_ref.shape[0] // 2
    A = A_ref[:N, :] + A_ref[NP:NP + N, :]
    alpha = alpha_ref[0, 0]
    consts = (FT_ref[...], Fsq_ref[...], cc_ref[...], C2bd_ref[...],
              logb_ref[...])
    _, G = _round_math(h_ref[...], dr_ref[...], A, alpha, consts, loga, invN,
                       want_T=False)
    P = Tin_ref[...] * G
    colsum = jnp.sum(P, axis=0, keepdims=True)
    r = lax.broadcasted_iota(jnp.int32, (TPL_W, N_TPL), 0) // TPL_N
    c = lax.broadcasted_iota(jnp.int32, (TPL_W, N_TPL), 1)
    sel = (r == c).astype(jnp.float32)
    fgw = jnp.dot(colsum, sel, preferred_element_type=jnp.float32)
    out_ref[...] = (
        jnp.dot(fgw, wlin_ref[...], preferred_element_type=jnp.float32)
        + blin_ref[...])


def kernel(x, edge_index, W_gcn, b_gcn, C_templates, F_templates,
           q_templates, alpha_logit, W_lin, b_lin):
    N, F = x.shape
    H = W_gcn.shape[1]
    E = edge_index.shape[1]
    rows = E // _CH

    src2 = edge_index[0].reshape(rows, _CH)
    dst2 = edge_index[1].reshape(rows, _CH)

    C2 = 0.5 * (C_templates + jnp.transpose(C_templates, (0, 2, 1)))
    b = jax.nn.softmax(q_templates, axis=1)
    logb = jnp.log(b).reshape(1, TPL_W)
    bflat = b.reshape(1, TPL_W)
    colconst = jnp.einsum('kij,kj->ki', C2 * C2, b).reshape(1, TPL_W)
    eye = jnp.eye(N_TPL, dtype=jnp.float32)
    C2bd = (eye[:, None, :, None] * C2[:, :, None, :]).reshape(TPL_W, TPL_W)
    FT = F_templates.reshape(TPL_W, H).T
    Fsq = jnp.sum(F_templates * F_templates, axis=2).reshape(1, TPL_W)
    alpha = jax.nn.sigmoid(alpha_logit).reshape(1, 1)
    loga = -float(np.log(N))
    invN = 1.0 / N
    carr = (FT, Fsq, colconst, C2bd, logb)

    npw = _padded_n(N) // _NS
    zeros128 = jnp.zeros((npw, TPL_W), jnp.float32)
    ones128 = jnp.ones((_CH, TPL_W), jnp.float32)

    cnt = _make_hist(E, N)(dst2, ones128, zeros128)

    h0, hs, deg_raw, dinv, invdeg = pl.pallas_call(
        _k1_body,
        out_shape=[
            jax.ShapeDtypeStruct((N, H), jnp.float32),
            jax.ShapeDtypeStruct((N, TPL_W), jnp.float32),
            jax.ShapeDtypeStruct((N, 1), jnp.float32),
            jax.ShapeDtypeStruct((N, 1), jnp.float32),
            jax.ShapeDtypeStruct((N, 1), jnp.float32),
        ],
    )(x, W_gcn, cnt)

    seg128 = _make_seg(E, N)
    S = seg128(hs, src2, dst2, zeros128)

    h, T = pl.pallas_call(
        functools.partial(_k2_body, loga, invN),
        out_shape=[
            jax.ShapeDtypeStruct((N, H), jnp.float32),
            jax.ShapeDtypeStruct((N, TPL_W), jnp.float32),
        ],
    )(h0, S, deg_raw, dinv, invdeg, b_gcn.reshape(1, H), bflat, alpha, *carr)

    kround = pl.pallas_call(
        functools.partial(_kround_body, loga, invN),
        out_shape=jax.ShapeDtypeStruct((N, TPL_W), jnp.float32),
    )

    for _ in range(2):
        A = seg128(T, src2, dst2, zeros128)
        T = kround(h, A, deg_raw, alpha, *carr)

    A = seg128(T, src2, dst2, zeros128)
    out = pl.pallas_call(
        functools.partial(_kfinal_body, loga, invN),
        out_shape=jax.ShapeDtypeStruct((1, W_lin.shape[1]), jnp.float32),
    )(h, A, T, deg_raw, alpha, *carr, W_lin, b_lin.reshape(1, -1))
    return out


# speedup vs baseline: 1.1030x; 1.1030x over previous
import functools

import jax
import jax.numpy as jnp
import numpy as np
from jax import lax
from jax.experimental import pallas as pl
from jax.experimental.pallas import tpu as pltpu
from jax.experimental.pallas import tpu_sc as plsc

EPS = 0.05
N_TPL = 8
TPL_N = 16
TPL_W = N_TPL * TPL_N

_NC = 2
_NS = 16
_NW = _NC * _NS

_CH = 125
_IB = 16


def _padded_n(N):
    q = _NS * 8
    return ((N + q - 1) // q) * q


def _make_hist(E, N):
    rows = E // _CH
    rpw = rows // _NW
    NP = _padded_n(N)
    npw = NP // _NS
    mesh = plsc.VectorSubcoreMesh(core_axis_name="c", subcore_axis_name="s")

    @functools.partial(
        pl.kernel, mesh=mesh,
        out_type=jax.ShapeDtypeStruct((2 * NP, 16), jnp.float32),
        scratch_types=[
            pltpu.VMEM((rpw, _CH), jnp.int32),
            pltpu.VMEM((_CH, 16), jnp.float32),
            pltpu.VMEM_SHARED((NP, 16), jnp.float32),
            pltpu.SemaphoreType.DMA,
        ],
        compiler_params=pltpu.CompilerParams(use_tc_tiling_on_sc=False),
    )
    def hist(dst_hbm, ones_hbm, zeros_hbm, out_hbm, dst_v, ones_v, acc_sh,
             ssem):
        cid = lax.axis_index("c")
        sid = lax.axis_index("s")
        wid = cid * _NS + sid
        pltpu.sync_copy(zeros_hbm.at[pl.ds(0, npw)],
                        acc_sh.at[pl.ds(sid * npw, npw)])
        pltpu.sync_copy(ones_hbm, ones_v)
        pltpu.sync_copy(dst_hbm.at[pl.ds(wid * rpw, rpw)], dst_v)
        plsc.subcore_barrier()

        def body(c, carry):
            @pl.when(c < rpw)
            def _():
                pltpu.async_copy(ones_v, acc_sh.at[dst_v.at[c]], ssem,
                                 add=True)

            @pl.when(c >= 8)
            def _():
                pltpu.make_async_copy(ones_v, acc_sh.at[dst_v.at[c - 8]],
                                      ssem).wait()

            return carry

        lax.fori_loop(0, rpw + 8, body, 0)
        plsc.subcore_barrier()
        pltpu.sync_copy(acc_sh.at[pl.ds(sid * npw, npw)],
                        out_hbm.at[pl.ds(cid * NP + sid * npw, npw)])

    return hist


def _make_seg(E, N, W, tc_tiling=True):
    rows = E // _CH
    rpw = rows // _NW
    NP = _padded_n(N)
    npw = NP // _NS
    mesh = plsc.VectorSubcoreMesh(core_axis_name="c", subcore_axis_name="s")

    assert rpw % _IB == 0

    @functools.partial(
        pl.kernel, mesh=mesh,
        out_type=jax.ShapeDtypeStruct((2 * NP, W), jnp.float32),
        scratch_types=[
            pltpu.VMEM((2, _IB, _CH), jnp.int32),
            pltpu.VMEM((2, _IB, _CH), jnp.int32),
            pltpu.VMEM((2, _CH, W), jnp.float32),
            pltpu.VMEM_SHARED((NP, W), jnp.float32),
            pltpu.SemaphoreType.DMA,
            pltpu.SemaphoreType.DMA,
            pltpu.SemaphoreType.DMA,
        ],
        compiler_params=pltpu.CompilerParams(use_tc_tiling_on_sc=tc_tiling),
    )
    def seg(vals_hbm, src_hbm, dst_hbm, zeros_hbm, out_hbm,
            src_v, dst_v, rows_v, acc_sh, gsem, isem, ssem):
        cid = lax.axis_index("c")
        sid = lax.axis_index("s")
        wid = cid * _NS + sid
        base = wid * rpw
        pltpu.sync_copy(src_hbm.at[pl.ds(base, _IB)], src_v.at[0])
        pltpu.sync_copy(dst_hbm.at[pl.ds(base, _IB)], dst_v.at[0])
        pltpu.sync_copy(zeros_hbm.at[pl.ds(0, npw)],
                        acc_sh.at[pl.ds(sid * npw, npw)])
        plsc.subcore_barrier()

        def body(c, carry):
            p = c % 2
            j = c // _IB
            jp = j % 2
            cm = c - 1
            pm = 1 - p
            jm = cm // _IB
            c2 = c - 2
            j2 = c2 // _IB

            @pl.when((c >= 1) & (c <= rpw))
            def _():
                pltpu.make_async_copy(
                    vals_hbm.at[src_v.at[jm % 2].at[cm % _IB]],
                    rows_v.at[pm], gsem).wait()

            @pl.when((c >= 2) & (c2 < rpw))
            def _():
                pltpu.make_async_copy(
                    rows_v.at[p],
                    acc_sh.at[dst_v.at[j2 % 2].at[c2 % _IB]], ssem).wait()

            @pl.when((c % _IB == 0) & (c > 0) & (c < rpw))
            def _():
                pltpu.make_async_copy(
                    src_hbm.at[pl.ds(base + j * _IB, _IB)],
                    src_v.at[jp], isem).wait()
                pltpu.make_async_copy(
                    dst_hbm.at[pl.ds(base + j * _IB, _IB)],
                    dst_v.at[jp], isem).wait()

            @pl.when(c < rpw)
            def _():
                pltpu.async_copy(
                    vals_hbm.at[src_v.at[jp].at[c % _IB]],
                    rows_v.at[p], gsem)

            @pl.when((c >= 1) & (c <= rpw))
            def _():
                pltpu.async_copy(rows_v.at[pm],
                                 acc_sh.at[dst_v.at[jm % 2].at[cm % _IB]],
                                 ssem, add=True)

            @pl.when((c % _IB == 1) & (c - 1 + _IB < rpw))
            def _():
                off = base + (j + 1) * _IB
                pltpu.async_copy(src_hbm.at[pl.ds(off, _IB)],
                                 src_v.at[1 - jp], isem)
                pltpu.async_copy(dst_hbm.at[pl.ds(off, _IB)],
                                 dst_v.at[1 - jp], isem)

            return carry

        lax.fori_loop(0, rpw + 2, body, 0)
        plsc.subcore_barrier()
        pltpu.sync_copy(acc_sh.at[pl.ds(sid * npw, npw)],
                        out_hbm.at[pl.ds(cid * NP + sid * npw, npw)])

    return seg


def _blockmax(row):
    parts = []
    for k in range(N_TPL):
        m = jnp.max(row[:, k * TPL_N:(k + 1) * TPL_N], axis=1, keepdims=True)
        parts.append(jnp.broadcast_to(m, (1, TPL_N)))
    return jnp.concatenate(parts, axis=1)


def _bblock():
    r = lax.broadcasted_iota(jnp.int32, (TPL_W, TPL_W), 0) // TPL_N
    c = lax.broadcasted_iota(jnp.int32, (TPL_W, TPL_W), 1) // TPL_N
    return (r == c).astype(jnp.float32)


def _round_math(h, deg_raw, A, alpha, consts, loga, invN, want_T=True):
    FT, Fsq, colconst, C2bd, logb = consts
    hsq = jnp.sum(h * h, axis=1, keepdims=True)
    M = hsq + Fsq - 2.0 * jnp.dot(h, FT, preferred_element_type=jnp.float32)
    cC = deg_raw * invN + colconst
    tens = cC - 2.0 * jnp.dot(A, C2bd, preferred_element_type=jnp.float32)
    G = (1.0 - alpha) * M + alpha * tens
    if not want_T:
        return None, G
    Bb = _bblock()
    mb = _blockmax(jnp.max(jnp.abs(G), axis=0, keepdims=True))
    negGn_eps = G * (-1.0 / (EPS * (mb + 1e-8)))
    gdiv = jnp.zeros((1, TPL_W), jnp.float32)
    fdiv = None
    for _ in range(5):
        X = gdiv + negGn_eps
        xm_b = _blockmax(jnp.max(X, axis=0, keepdims=True))
        Ex = jnp.exp(X - xm_b)
        Sb = jnp.dot(Ex, Bb, preferred_element_type=jnp.float32)
        fdiv = loga - xm_b - jnp.log(Sb)
        Y = fdiv + negGn_eps
        ym = jnp.max(Y, axis=0, keepdims=True)
        Sg = jnp.sum(jnp.exp(Y - ym), axis=0, keepdims=True)
        gdiv = logb - ym - jnp.log(Sg)
    T = jnp.exp(fdiv + gdiv + negGn_eps)
    return T, G


def _k1_body(x_ref, w_ref, cnt_ref, h0_ref, hs_ref, dr_ref, dinv_ref, idg_ref):
    N = x_ref.shape[0]
    H = w_ref.shape[1]
    NP = cnt_ref.shape[0] // 2
    h0 = jnp.dot(x_ref[...], w_ref[...], preferred_element_type=jnp.float32)
    cnt = cnt_ref[...]
    deg_raw = cnt[:N, 0:1] + cnt[NP:NP + N, 0:1]
    deg = deg_raw + 1.0
    dinv = lax.rsqrt(deg)
    h0_ref[...] = h0
    hs_ref[...] = h0 * dinv
    dr_ref[...] = deg_raw
    dinv_ref[...] = dinv
    idg_ref[...] = 1.0 / deg


def _k2_body(loga, invN, h0_ref, S_ref, dr_ref, dinv_ref, idg_ref, bg_ref,
             bflat_ref, alpha_ref, FT_ref, Fsq_ref, cc_ref, C2bd_ref,
             logb_ref, h_ref, T_ref):
    N = h0_ref.shape[0]
    H = h0_ref.shape[1]
    NP = S_ref.shape[0] // 2
    S = S_ref[:N, :] + S_ref[NP:NP + N, :]
    h = dinv_ref[...] * S + h0_ref[...] * idg_ref[...] + bg_ref[...]
    h_ref[...] = h
    deg_raw = dr_ref[...]
    alpha = alpha_ref[0, 0]
    consts = (FT_ref[...], Fsq_ref[...], cc_ref[...], C2bd_ref[...],
              logb_ref[...])
    A0 = deg_raw * invN * bflat_ref[...]
    T, _ = _round_math(h, deg_raw, A0, alpha, consts, loga, invN, want_T=True)
    T_ref[...] = T


def _kround_body(loga, invN, h_ref, A_ref, dr_ref, alpha_ref, FT_ref,
                 Fsq_ref, cc_ref, C2bd_ref, logb_ref, T_ref):
    N = h_ref.shape[0]
    NP = A_ref.shape[0] // 2
    A = A_ref[:N, :] + A_ref[NP:NP + N, :]
    alpha = alpha_ref[0, 0]
    consts = (FT_ref[...], Fsq_ref[...], cc_ref[...], C2bd_ref[...],
              logb_ref[...])
    T, _ = _round_math(h_ref[...], dr_ref[...], A, alpha, consts, loga, invN,
                       want_T=True)
    T_ref[...] = T


def _kfinal_body(loga, invN, h_ref, A_ref, Tin_ref, dr_ref, alpha_ref,
                 FT_ref, Fsq_ref, cc_ref, C2bd_ref, logb_ref,
                 wlin_ref, blin_ref, out_ref):
    N = h_ref.shape[0]
    NP = A_ref.shape[0] // 2
    A = A_ref[:N, :] + A_ref[NP:NP + N, :]
    alpha = alpha_ref[0, 0]
    consts = (FT_ref[...], Fsq_ref[...], cc_ref[...], C2bd_ref[...],
              logb_ref[...])
    _, G = _round_math(h_ref[...], dr_ref[...], A, alpha, consts, loga, invN,
                       want_T=False)
    P = Tin_ref[...] * G
    colsum = jnp.sum(P, axis=0, keepdims=True)
    r = lax.broadcasted_iota(jnp.int32, (TPL_W, N_TPL), 0) // TPL_N
    c = lax.broadcasted_iota(jnp.int32, (TPL_W, N_TPL), 1)
    sel = (r == c).astype(jnp.float32)
    fgw = jnp.dot(colsum, sel, preferred_element_type=jnp.float32)
    out_ref[...] = (
        jnp.dot(fgw, wlin_ref[...], preferred_element_type=jnp.float32)
        + blin_ref[...])


def kernel(x, edge_index, W_gcn, b_gcn, C_templates, F_templates,
           q_templates, alpha_logit, W_lin, b_lin):
    N, F = x.shape
    H = W_gcn.shape[1]
    E = edge_index.shape[1]
    rows = E // _CH

    src2 = edge_index[0].reshape(rows, _CH)
    dst2 = edge_index[1].reshape(rows, _CH)

    C2 = 0.5 * (C_templates + jnp.transpose(C_templates, (0, 2, 1)))
    b = jax.nn.softmax(q_templates, axis=1)
    logb = jnp.log(b).reshape(1, TPL_W)
    bflat = b.reshape(1, TPL_W)
    colconst = jnp.einsum('kij,kj->ki', C2 * C2, b).reshape(1, TPL_W)
    eye = jnp.eye(N_TPL, dtype=jnp.float32)
    C2bd = (eye[:, None, :, None] * C2[:, :, None, :]).reshape(TPL_W, TPL_W)
    FT = F_templates.reshape(TPL_W, H).T
    Fsq = jnp.sum(F_templates * F_templates, axis=2).reshape(1, TPL_W)
    alpha = jax.nn.sigmoid(alpha_logit).reshape(1, 1)
    loga = -float(np.log(N))
    invN = 1.0 / N
    carr = (FT, Fsq, colconst, C2bd, logb)

    npw = _padded_n(N) // _NS
    zeros128 = jnp.zeros((npw, TPL_W), jnp.float32)
    ones16 = jnp.ones((_CH, 16), jnp.float32)
    zeros16 = jnp.zeros((npw, 16), jnp.float32)

    cnt = _make_hist(E, N)(dst2, ones16, zeros16)

    h0, hs, deg_raw, dinv, invdeg = pl.pallas_call(
        _k1_body,
        out_shape=[
            jax.ShapeDtypeStruct((N, H), jnp.float32),
            jax.ShapeDtypeStruct((N, H), jnp.float32),
            jax.ShapeDtypeStruct((N, 1), jnp.float32),
            jax.ShapeDtypeStruct((N, 1), jnp.float32),
            jax.ShapeDtypeStruct((N, 1), jnp.float32),
        ],
    )(x, W_gcn, cnt)

    zeros64 = jnp.zeros((npw, H), jnp.float32)
    S = _make_seg(E, N, H, tc_tiling=False)(hs, src2, dst2, zeros64)
    seg128 = _make_seg(E, N, TPL_W)

    h, T = pl.pallas_call(
        functools.partial(_k2_body, loga, invN),
        out_shape=[
            jax.ShapeDtypeStruct((N, H), jnp.float32),
            jax.ShapeDtypeStruct((N, TPL_W), jnp.float32),
        ],
    )(h0, S, deg_raw, dinv, invdeg, b_gcn.reshape(1, H), bflat, alpha, *carr)

    kround = pl.pallas_call(
        functools.partial(_kround_body, loga, invN),
        out_shape=jax.ShapeDtypeStruct((N, TPL_W), jnp.float32),
    )

    for _ in range(2):
        A = seg128(T, src2, dst2, zeros128)
        T = kround(h, A, deg_raw, alpha, *carr)

    A = seg128(T, src2, dst2, zeros128)
    out = pl.pallas_call(
        functools.partial(_kfinal_body, loga, invN),
        out_shape=jax.ShapeDtypeStruct((1, W_lin.shape[1]), jnp.float32),
    )(h, A, T, deg_raw, alpha, *carr, W_lin, b_lin.reshape(1, -1))
    return out


# speedup vs baseline: 1.1603x; 1.0520x over previous
import functools

import jax
import jax.numpy as jnp
import numpy as np
from jax import lax
from jax.experimental import pallas as pl
from jax.experimental.pallas import tpu as pltpu
from jax.experimental.pallas import tpu_sc as plsc

EPS = 0.05
N_TPL = 8
TPL_N = 16
TPL_W = N_TPL * TPL_N

_NC = 2
_NS = 16
_NW = _NC * _NS

_CH = 125
_IB = 16


def _padded_n(N):
    q = _NS * 8
    return ((N + q - 1) // q) * q


def _make_hist(E, N):
    rows = E // _CH
    rpw = rows // _NW
    NP = _padded_n(N)
    npw = NP // _NS
    mesh = plsc.VectorSubcoreMesh(core_axis_name="c", subcore_axis_name="s")

    @functools.partial(
        pl.kernel, mesh=mesh,
        out_type=jax.ShapeDtypeStruct((2 * NP, 16), jnp.float32),
        scratch_types=[
            pltpu.VMEM((rpw, _CH), jnp.int32),
            pltpu.VMEM((_CH, 16), jnp.float32),
            pltpu.VMEM_SHARED((NP, 16), jnp.float32),
            pltpu.SemaphoreType.DMA,
        ],
        compiler_params=pltpu.CompilerParams(use_tc_tiling_on_sc=False),
    )
    def hist(dst_hbm, ones_hbm, zeros_hbm, out_hbm, dst_v, ones_v, acc_sh,
             ssem):
        cid = lax.axis_index("c")
        sid = lax.axis_index("s")
        wid = cid * _NS + sid
        pltpu.sync_copy(zeros_hbm.at[pl.ds(0, npw)],
                        acc_sh.at[pl.ds(sid * npw, npw)])
        pltpu.sync_copy(ones_hbm, ones_v)
        pltpu.sync_copy(dst_hbm.at[pl.ds(wid * rpw, rpw)], dst_v)
        plsc.subcore_barrier()

        def body(c, carry):
            @pl.when(c < rpw)
            def _():
                pltpu.async_copy(ones_v, acc_sh.at[dst_v.at[c]], ssem,
                                 add=True)

            @pl.when(c >= 8)
            def _():
                pltpu.make_async_copy(ones_v, acc_sh.at[dst_v.at[c - 8]],
                                      ssem).wait()

            return carry

        lax.fori_loop(0, rpw + 8, body, 0)
        plsc.subcore_barrier()
        pltpu.sync_copy(acc_sh.at[pl.ds(sid * npw, npw)],
                        out_hbm.at[pl.ds(cid * NP + sid * npw, npw)])

    return hist


def _make_seg(E, N, W, tc_tiling=True):
    rows = E // _CH
    rpw = rows // _NW
    NP = _padded_n(N)
    npw = NP // _NS
    mesh = plsc.VectorSubcoreMesh(core_axis_name="c", subcore_axis_name="s")

    assert rpw % _IB == 0

    @functools.partial(
        pl.kernel, mesh=mesh,
        out_type=jax.ShapeDtypeStruct((2 * NP, W), jnp.float32),
        scratch_types=[
            pltpu.VMEM((2, _IB, _CH), jnp.int32),
            pltpu.VMEM((2, _IB, _CH), jnp.int32),
            pltpu.VMEM((2, _CH, W), jnp.float32),
            pltpu.VMEM_SHARED((NP, W), jnp.float32),
            pltpu.SemaphoreType.DMA,
            pltpu.SemaphoreType.DMA,
            pltpu.SemaphoreType.DMA,
        ],
        compiler_params=pltpu.CompilerParams(use_tc_tiling_on_sc=tc_tiling),
    )
    def seg(vals_hbm, src_hbm, dst_hbm, zeros_hbm, out_hbm,
            src_v, dst_v, rows_v, acc_sh, gsem, isem, ssem):
        cid = lax.axis_index("c")
        sid = lax.axis_index("s")
        wid = cid * _NS + sid
        base = wid * rpw
        pltpu.sync_copy(src_hbm.at[pl.ds(base, _IB)], src_v.at[0])
        pltpu.sync_copy(dst_hbm.at[pl.ds(base, _IB)], dst_v.at[0])
        pltpu.sync_copy(zeros_hbm.at[pl.ds(0, npw)],
                        acc_sh.at[pl.ds(sid * npw, npw)])
        plsc.subcore_barrier()

        def body(c, carry):
            p = c % 2
            j = c // _IB
            jp = j % 2
            cm = c - 1
            pm = 1 - p
            jm = cm // _IB
            c2 = c - 2
            j2 = c2 // _IB

            @pl.when((c >= 1) & (c <= rpw))
            def _():
                pltpu.make_async_copy(
                    vals_hbm.at[src_v.at[jm % 2].at[cm % _IB]],
                    rows_v.at[pm], gsem).wait()

            @pl.when((c >= 2) & (c2 < rpw))
            def _():
                pltpu.make_async_copy(
                    rows_v.at[p],
                    acc_sh.at[dst_v.at[j2 % 2].at[c2 % _IB]], ssem).wait()

            @pl.when((c % _IB == 0) & (c > 0) & (c < rpw))
            def _():
                pltpu.make_async_copy(
                    src_hbm.at[pl.ds(base + j * _IB, _IB)],
                    src_v.at[jp], isem).wait()
                pltpu.make_async_copy(
                    dst_hbm.at[pl.ds(base + j * _IB, _IB)],
                    dst_v.at[jp], isem).wait()

            @pl.when(c < rpw)
            def _():
                pltpu.async_copy(
                    vals_hbm.at[src_v.at[jp].at[c % _IB]],
                    rows_v.at[p], gsem)

            @pl.when((c >= 1) & (c <= rpw))
            def _():
                pltpu.async_copy(rows_v.at[pm],
                                 acc_sh.at[dst_v.at[jm % 2].at[cm % _IB]],
                                 ssem, add=True)

            @pl.when((c % _IB == 1) & (c - 1 + _IB < rpw))
            def _():
                off = base + (j + 1) * _IB
                pltpu.async_copy(src_hbm.at[pl.ds(off, _IB)],
                                 src_v.at[1 - jp], isem)
                pltpu.async_copy(dst_hbm.at[pl.ds(off, _IB)],
                                 dst_v.at[1 - jp], isem)

            return carry

        lax.fori_loop(0, rpw + 2, body, 0)
        plsc.subcore_barrier()
        pltpu.sync_copy(acc_sh.at[pl.ds(sid * npw, npw)],
                        out_hbm.at[pl.ds(cid * NP + sid * npw, npw)])

    return seg


def _blockmax(row):
    parts = []
    for k in range(N_TPL):
        m = jnp.max(row[:, k * TPL_N:(k + 1) * TPL_N], axis=1, keepdims=True)
        parts.append(jnp.broadcast_to(m, (1, TPL_N)))
    return jnp.concatenate(parts, axis=1)


def _bblock():
    r = lax.broadcasted_iota(jnp.int32, (TPL_W, TPL_W), 0) // TPL_N
    c = lax.broadcasted_iota(jnp.int32, (TPL_W, TPL_W), 1) // TPL_N
    return (r == c).astype(jnp.float32)


def _round_math(h, deg_raw, A, alpha, consts, loga, invN, want_T=True):
    FT, Fsq, colconst, C2bd, logb = consts
    hsq = jnp.sum(h * h, axis=1, keepdims=True)
    M = hsq + Fsq - 2.0 * jnp.dot(h, FT, preferred_element_type=jnp.float32)
    cC = deg_raw * invN + colconst
    tens = cC - 2.0 * jnp.dot(A, C2bd, preferred_element_type=jnp.float32)
    G = (1.0 - alpha) * M + alpha * tens
    if not want_T:
        return None, G
    Bb = _bblock()
    mb = _blockmax(jnp.max(jnp.abs(G), axis=0, keepdims=True))
    negGn_eps = G * (-1.0 / (EPS * (mb + 1e-8)))
    W0 = jnp.exp(negGn_eps)
    negmax = jnp.max(negGn_eps, axis=0, keepdims=True)
    gdiv = jnp.zeros((1, TPL_W), jnp.float32)
    R = None
    xm_b = None
    for _ in range(5):
        xm_b = _blockmax(gdiv + negmax)
        rc = jnp.exp(gdiv - xm_b)
        Sb = jnp.dot(W0, rc.reshape(TPL_W, 1) * Bb,
                     preferred_element_type=jnp.float32)
        R = W0 / Sb
        S2 = jnp.sum(R, axis=0, keepdims=True)
        gdiv = logb - loga + xm_b - jnp.log(S2)
    T = jnp.exp(loga - xm_b + gdiv) * R
    return T, G


def _k1_body(x_ref, w_ref, cnt_ref, h0_ref, hs_ref, dr_ref, dinv_ref, idg_ref):
    N = x_ref.shape[0]
    H = w_ref.shape[1]
    NP = cnt_ref.shape[0] // 2
    h0 = jnp.dot(x_ref[...], w_ref[...], preferred_element_type=jnp.float32)
    cnt = cnt_ref[...]
    deg_raw = cnt[:N, 0:1] + cnt[NP:NP + N, 0:1]
    deg = deg_raw + 1.0
    dinv = lax.rsqrt(deg)
    h0_ref[...] = h0
    hs_ref[...] = h0 * dinv
    dr_ref[...] = deg_raw
    dinv_ref[...] = dinv
    idg_ref[...] = 1.0 / deg


def _k2_body(loga, invN, h0_ref, S_ref, dr_ref, dinv_ref, idg_ref, bg_ref,
             bflat_ref, alpha_ref, FT_ref, Fsq_ref, cc_ref, C2bd_ref,
             logb_ref, h_ref, T_ref):
    N = h0_ref.shape[0]
    H = h0_ref.shape[1]
    NP = S_ref.shape[0] // 2
    S = S_ref[:N, :] + S_ref[NP:NP + N, :]
    h = dinv_ref[...] * S + h0_ref[...] * idg_ref[...] + bg_ref[...]
    h_ref[...] = h
    deg_raw = dr_ref[...]
    alpha = alpha_ref[0, 0]
    consts = (FT_ref[...], Fsq_ref[...], cc_ref[...], C2bd_ref[...],
              logb_ref[...])
    A0 = deg_raw * invN * bflat_ref[...]
    T, _ = _round_math(h, deg_raw, A0, alpha, consts, loga, invN, want_T=True)
    T_ref[...] = T


def _kround_body(loga, invN, h_ref, A_ref, dr_ref, alpha_ref, FT_ref,
                 Fsq_ref, cc_ref, C2bd_ref, logb_ref, T_ref):
    N = h_ref.shape[0]
    NP = A_ref.shape[0] // 2
    A = A_ref[:N, :] + A_ref[NP:NP + N, :]
    alpha = alpha_ref[0, 0]
    consts = (FT_ref[...], Fsq_ref[...], cc_ref[...], C2bd_ref[...],
              logb_ref[...])
    T, _ = _round_math(h_ref[...], dr_ref[...], A, alpha, consts, loga, invN,
                       want_T=True)
    T_ref[...] = T


def _kfinal_body(loga, invN, h_ref, A_ref, Tin_ref, dr_ref, alpha_ref,
                 FT_ref, Fsq_ref, cc_ref, C2bd_ref, logb_ref,
                 wlin_ref, blin_ref, out_ref):
    N = h_ref.shape[0]
    NP = A_ref.shape[0] // 2
    A = A_ref[:N, :] + A_ref[NP:NP + N, :]
    alpha = alpha_ref[0, 0]
    consts = (FT_ref[...], Fsq_ref[...], cc_ref[...], C2bd_ref[...],
              logb_ref[...])
    _, G = _round_math(h_ref[...], dr_ref[...], A, alpha, consts, loga, invN,
                       want_T=False)
    P = Tin_ref[...] * G
    colsum = jnp.sum(P, axis=0, keepdims=True)
    r = lax.broadcasted_iota(jnp.int32, (TPL_W, N_TPL), 0) // TPL_N
    c = lax.broadcasted_iota(jnp.int32, (TPL_W, N_TPL), 1)
    sel = (r == c).astype(jnp.float32)
    fgw = jnp.dot(colsum, sel, preferred_element_type=jnp.float32)
    out_ref[...] = (
        jnp.dot(fgw, wlin_ref[...], preferred_element_type=jnp.float32)
        + blin_ref[...])


def kernel(x, edge_index, W_gcn, b_gcn, C_templates, F_templates,
           q_templates, alpha_logit, W_lin, b_lin):
    N, F = x.shape
    H = W_gcn.shape[1]
    E = edge_index.shape[1]
    rows = E // _CH

    src2 = edge_index[0].reshape(rows, _CH)
    dst2 = edge_index[1].reshape(rows, _CH)

    C2 = 0.5 * (C_templates + jnp.transpose(C_templates, (0, 2, 1)))
    b = jax.nn.softmax(q_templates, axis=1)
    logb = jnp.log(b).reshape(1, TPL_W)
    bflat = b.reshape(1, TPL_W)
    colconst = jnp.einsum('kij,kj->ki', C2 * C2, b).reshape(1, TPL_W)
    eye = jnp.eye(N_TPL, dtype=jnp.float32)
    C2bd = (eye[:, None, :, None] * C2[:, :, None, :]).reshape(TPL_W, TPL_W)
    FT = F_templates.reshape(TPL_W, H).T
    Fsq = jnp.sum(F_templates * F_templates, axis=2).reshape(1, TPL_W)
    alpha = jax.nn.sigmoid(alpha_logit).reshape(1, 1)
    loga = -float(np.log(N))
    invN = 1.0 / N
    carr = (FT, Fsq, colconst, C2bd, logb)

    npw = _padded_n(N) // _NS
    zeros128 = jnp.zeros((npw, TPL_W), jnp.float32)
    ones16 = jnp.ones((_CH, 16), jnp.float32)
    zeros16 = jnp.zeros((npw, 16), jnp.float32)

    cnt = _make_hist(E, N)(dst2, ones16, zeros16)

    h0, hs, deg_raw, dinv, invdeg = pl.pallas_call(
        _k1_body,
        out_shape=[
            jax.ShapeDtypeStruct((N, H), jnp.float32),
            jax.ShapeDtypeStruct((N, H), jnp.float32),
            jax.ShapeDtypeStruct((N, 1), jnp.float32),
            jax.ShapeDtypeStruct((N, 1), jnp.float32),
            jax.ShapeDtypeStruct((N, 1), jnp.float32),
        ],
    )(x, W_gcn, cnt)

    zeros64 = jnp.zeros((npw, H), jnp.float32)
    S = _make_seg(E, N, H, tc_tiling=False)(hs, src2, dst2, zeros64)
    seg128 = _make_seg(E, N, TPL_W)

    h, T = pl.pallas_call(
        functools.partial(_k2_body, loga, invN),
        out_shape=[
            jax.ShapeDtypeStruct((N, H), jnp.float32),
            jax.ShapeDtypeStruct((N, TPL_W), jnp.float32),
        ],
    )(h0, S, deg_raw, dinv, invdeg, b_gcn.reshape(1, H), bflat, alpha, *carr)

    kround = pl.pallas_call(
        functools.partial(_kround_body, loga, invN),
        out_shape=jax.ShapeDtypeStruct((N, TPL_W), jnp.float32),
    )

    for _ in range(2):
        A = seg128(T, src2, dst2, zeros128)
        T = kround(h, A, deg_raw, alpha, *carr)

    A = seg128(T, src2, dst2, zeros128)
    out = pl.pallas_call(
        functools.partial(_kfinal_body, loga, invN),
        out_shape=jax.ShapeDtypeStruct((1, W_lin.shape[1]), jnp.float32),
    )(h, A, T, deg_raw, alpha, *carr, W_lin, b_lin.reshape(1, -1))
    return out
